# initial kernel scaffold (unmeasured)
import functools

import jax
import jax.numpy as jnp
from jax import lax
from jax.experimental import pallas as pl
from jax.experimental.pallas import tpu as pltpu

N_DEV = 4
M_BLK = 2048
K_PER = 2048
K_GLB = 8192
N_OUT = 4096


def _a2a_body(x_ref, out_ref, local_sem, send_sems, recv_sems):
    my = lax.axis_index("i")

    bar = pltpu.get_barrier_semaphore()
    for off in (1, 2, 3):
        pl.semaphore_signal(
            bar, inc=1,
            device_id=((my + off) % N_DEV,),
            device_id_type=pl.DeviceIdType.MESH,
        )
    pl.semaphore_wait(bar, N_DEV - 1)

    local_copy = pltpu.make_async_copy(
        x_ref.at[pl.ds(my * M_BLK, M_BLK), :],
        out_ref.at[:, pl.ds(my * K_PER, K_PER)],
        local_sem,
    )
    local_copy.start()

    sends = []
    for off in (1, 2, 3):
        dst = (my + off) % N_DEV
        rdma = pltpu.make_async_remote_copy(
            src_ref=x_ref.at[pl.ds(dst * M_BLK, M_BLK), :],
            dst_ref=out_ref.at[:, pl.ds(my * K_PER, K_PER)],
            send_sem=send_sems.at[off - 1],
            recv_sem=recv_sems.at[off - 1],
            device_id=(dst,),
            device_id_type=pl.DeviceIdType.MESH,
        )
        rdma.start()
        sends.append(rdma)

    for off in (1, 2, 3):
        src = (my - off) % N_DEV
        recv = pltpu.make_async_remote_copy(
            src_ref=x_ref.at[pl.ds(0, M_BLK), :],
            dst_ref=out_ref.at[:, pl.ds(src * K_PER, K_PER)],
            send_sem=send_sems.at[off - 1],
            recv_sem=recv_sems.at[off - 1],
            device_id=(src,),
            device_id_type=pl.DeviceIdType.MESH,
        )
        recv.wait_recv()

    for rdma in sends:
        rdma.wait_send()
    local_copy.wait()


def _a2a(x_shard):
    return pl.pallas_call(
        _a2a_body,
        out_shape=jax.ShapeDtypeStruct((M_BLK, K_GLB), x_shard.dtype),
        in_specs=[pl.BlockSpec(memory_space=pltpu.ANY)],
        out_specs=pl.BlockSpec(memory_space=pltpu.ANY),
        scratch_shapes=[
            pltpu.SemaphoreType.DMA,
            pltpu.SemaphoreType.DMA((3,)),
            pltpu.SemaphoreType.DMA((3,)),
        ],
        compiler_params=pltpu.CompilerParams(collective_id=0),
    )(x_shard)


BM, BK, BN = 512, 2048, 1024
NM, NK, NN = M_BLK // BM, K_GLB // BK, N_OUT // BN


def _mm_body(xg_ref, w_ref, o_ref, acc_ref):
    k = pl.program_id(2)

    @pl.when(k == 0)
    def _():
        acc_ref[...] = jnp.zeros_like(acc_ref)

    acc_ref[...] += jnp.dot(
        xg_ref[...], w_ref[...], preferred_element_type=jnp.float32
    )

    @pl.when(k == NK - 1)
    def _():
        o_ref[...] = jnp.maximum(acc_ref[...], 0.0)


def _gemm_relu(xg, w_mat):
    return pl.pallas_call(
        _mm_body,
        grid=(NM, NN, NK),
        in_specs=[
            pl.BlockSpec((BM, BK), lambda i, j, k: (i, k)),
            pl.BlockSpec((BK, BN), lambda i, j, k: (k, j)),
        ],
        out_specs=pl.BlockSpec((BM, BN), lambda i, j, k: (i, j)),
        out_shape=jax.ShapeDtypeStruct((M_BLK, N_OUT), jnp.float32),
        scratch_shapes=[pltpu.VMEM((BM, BN), jnp.float32)],
        compiler_params=pltpu.CompilerParams(
            dimension_semantics=("parallel", "parallel", "arbitrary"),
        ),
    )(xg, w_mat)


def kernel(x, w_mat):
    xg = _a2a(x)
    return _gemm_relu(xg, w_mat)


# baseline (device time: 774063 ns/iter reference)
import functools

import jax
import jax.numpy as jnp
from jax import lax
from jax.experimental import pallas as pl
from jax.experimental.pallas import tpu as pltpu

N_DEV = 4
M_BLK = 2048
K_PER = 2048
K_GLB = 8192
N_OUT = 4096


def _a2a_body(x_ref, out_ref, local_sem, send_sems, recv_sems):
    my = lax.axis_index("i")

    bar = pltpu.get_barrier_semaphore()
    for off in (1, 2, 3):
        pl.semaphore_signal(
            bar, inc=1,
            device_id=((my + off) % N_DEV,),
            device_id_type=pl.DeviceIdType.MESH,
        )
    pl.semaphore_wait(bar, N_DEV - 1)

    local_copy = pltpu.make_async_copy(
        x_ref.at[pl.ds(my * M_BLK, M_BLK), :],
        out_ref.at[:, pl.ds(my * K_PER, K_PER)],
        local_sem,
    )
    local_copy.start()

    sends = []
    for off in (1, 2, 3):
        dst = (my + off) % N_DEV
        rdma = pltpu.make_async_remote_copy(
            src_ref=x_ref.at[pl.ds(dst * M_BLK, M_BLK), :],
            dst_ref=out_ref.at[:, pl.ds(my * K_PER, K_PER)],
            send_sem=send_sems.at[off - 1],
            recv_sem=recv_sems.at[off - 1],
            device_id=(dst,),
            device_id_type=pl.DeviceIdType.MESH,
        )
        rdma.start()
        sends.append(rdma)

    for off in (1, 2, 3):
        src = (my - off) % N_DEV
        recv = pltpu.make_async_remote_copy(
            src_ref=x_ref.at[pl.ds(0, M_BLK), :],
            dst_ref=out_ref.at[:, pl.ds(src * K_PER, K_PER)],
            send_sem=send_sems.at[off - 1],
            recv_sem=recv_sems.at[off - 1],
            device_id=(src,),
            device_id_type=pl.DeviceIdType.MESH,
        )
        recv.wait_recv()

    for rdma in sends:
        rdma.wait_send()
    local_copy.wait()


def _a2a(x_shard):
    return pl.pallas_call(
        _a2a_body,
        out_shape=jax.ShapeDtypeStruct((M_BLK, K_GLB), x_shard.dtype),
        in_specs=[pl.BlockSpec(memory_space=pl.ANY)],
        out_specs=pl.BlockSpec(memory_space=pl.ANY),
        scratch_shapes=[
            pltpu.SemaphoreType.DMA,
            pltpu.SemaphoreType.DMA((3,)),
            pltpu.SemaphoreType.DMA((3,)),
        ],
        compiler_params=pltpu.CompilerParams(collective_id=0),
    )(x_shard)


BM, BK, BN = 512, 2048, 1024
NM, NK, NN = M_BLK // BM, K_GLB // BK, N_OUT // BN


def _mm_body(xg_ref, w_ref, o_ref, acc_ref):
    k = pl.program_id(2)

    @pl.when(k == 0)
    def _():
        acc_ref[...] = jnp.zeros_like(acc_ref)

    acc_ref[...] += jnp.dot(
        xg_ref[...], w_ref[...], preferred_element_type=jnp.float32
    )

    @pl.when(k == NK - 1)
    def _():
        o_ref[...] = jnp.maximum(acc_ref[...], 0.0)


def _gemm_relu(xg, w_mat):
    return pl.pallas_call(
        _mm_body,
        grid=(NM, NN, NK),
        in_specs=[
            pl.BlockSpec((BM, BK), lambda i, j, k: (i, k)),
            pl.BlockSpec((BK, BN), lambda i, j, k: (k, j)),
        ],
        out_specs=pl.BlockSpec((BM, BN), lambda i, j, k: (i, j)),
        out_shape=jax.ShapeDtypeStruct((M_BLK, N_OUT), jnp.float32),
        scratch_shapes=[pltpu.VMEM((BM, BN), jnp.float32)],
        compiler_params=pltpu.CompilerParams(
            dimension_semantics=("parallel", "parallel", "arbitrary"),
        ),
    )(xg, w_mat)


def kernel(x, w_mat):
    xg = _a2a(x)
    return _gemm_relu(xg, w_mat)


# device time: 348416 ns/iter; 2.2217x vs baseline; 2.2217x over previous
import functools

import jax
import jax.numpy as jnp
from jax import lax
from jax.experimental import pallas as pl
from jax.experimental.pallas import tpu as pltpu

N_DEV = 4
M_BLK = 2048
K_PER = 2048
K_GLB = 8192
N_OUT = 4096


def _a2a_body(x_ref, out_ref, local_sem, send_sems, recv_sems):
    my = lax.axis_index("i")

    bar = pltpu.get_barrier_semaphore()
    for off in (1, 2, 3):
        pl.semaphore_signal(
            bar, inc=1,
            device_id=((my + off) % N_DEV,),
            device_id_type=pl.DeviceIdType.MESH,
        )
    pl.semaphore_wait(bar, N_DEV - 1)

    local_copy = pltpu.make_async_copy(
        x_ref.at[pl.ds(my * M_BLK, M_BLK), :],
        out_ref.at[:, pl.ds(my * K_PER, K_PER)],
        local_sem,
    )
    local_copy.start()

    sends = []
    for off in (1, 2, 3):
        dst = (my + off) % N_DEV
        rdma = pltpu.make_async_remote_copy(
            src_ref=x_ref.at[pl.ds(dst * M_BLK, M_BLK), :],
            dst_ref=out_ref.at[:, pl.ds(my * K_PER, K_PER)],
            send_sem=send_sems.at[off - 1],
            recv_sem=recv_sems.at[off - 1],
            device_id=(dst,),
            device_id_type=pl.DeviceIdType.MESH,
        )
        rdma.start()
        sends.append(rdma)

    for off in (1, 2, 3):
        src = (my - off) % N_DEV
        recv = pltpu.make_async_remote_copy(
            src_ref=x_ref.at[pl.ds(0, M_BLK), :],
            dst_ref=out_ref.at[:, pl.ds(src * K_PER, K_PER)],
            send_sem=send_sems.at[off - 1],
            recv_sem=recv_sems.at[off - 1],
            device_id=(src,),
            device_id_type=pl.DeviceIdType.MESH,
        )
        recv.wait_recv()

    for rdma in sends:
        rdma.wait_send()
    local_copy.wait()


def _a2a(x_shard):
    return pl.pallas_call(
        _a2a_body,
        out_shape=jax.ShapeDtypeStruct((M_BLK, K_GLB), x_shard.dtype),
        in_specs=[pl.BlockSpec(memory_space=pl.ANY)],
        out_specs=pl.BlockSpec(memory_space=pl.ANY),
        scratch_shapes=[
            pltpu.SemaphoreType.DMA,
            pltpu.SemaphoreType.DMA((3,)),
            pltpu.SemaphoreType.DMA((3,)),
        ],
        compiler_params=pltpu.CompilerParams(collective_id=0),
    )(x_shard)


BM, BK, BN = 512, 2048, 1024
NM, NK, NN = M_BLK // BM, K_GLB // BK, N_OUT // BN


def _mm_body(xg_ref, w_ref, o_ref, acc_ref):
    k = pl.program_id(2)

    @pl.when(k == 0)
    def _():
        acc_ref[...] = jnp.zeros_like(acc_ref)

    acc_ref[...] += jnp.dot(
        xg_ref[...], w_ref[...], preferred_element_type=jnp.float32
    )

    @pl.when(k == NK - 1)
    def _():
        o_ref[...] = jnp.maximum(acc_ref[...], 0.0)


def _gemm_relu(xg, w_mat):
    return pl.pallas_call(
        _mm_body,
        grid=(NM, NN, NK),
        in_specs=[
            pl.BlockSpec((BM, BK), lambda i, j, k: (i, k)),
            pl.BlockSpec((BK, BN), lambda i, j, k: (k, j)),
        ],
        out_specs=pl.BlockSpec((BM, BN), lambda i, j, k: (i, j)),
        out_shape=jax.ShapeDtypeStruct((M_BLK, N_OUT), jnp.float32),
        scratch_shapes=[pltpu.VMEM((BM, BN), jnp.float32)],
        compiler_params=pltpu.CompilerParams(
            dimension_semantics=("parallel", "parallel", "arbitrary"),
        ),
    )(xg, w_mat)


BN2 = 256
NN2 = N_OUT // BN2
_OFF_TO_SLOT = ((1, 1), (2, 3), (3, 2))


def _fused_body(perm_ref, x16_ref, w_ref, yin_ref, o_ref, xg_ref,
                local_sem, send_sems, recv_sems):
    p = pl.program_id(0)
    j = pl.program_id(1)
    my = lax.axis_index("i")

    @pl.when((p == 0) & (j == 0))
    def _():
        bar = pltpu.get_barrier_semaphore()
        for off in (1, 2, 3):
            pl.semaphore_signal(
                bar, inc=1,
                device_id=((my + off) % N_DEV,),
                device_id_type=pl.DeviceIdType.MESH,
            )
        pl.semaphore_wait(bar, N_DEV - 1)

        local_copy = pltpu.make_async_copy(
            x16_ref.at[pl.ds(my * M_BLK, M_BLK), :], xg_ref.at[0], local_sem
        )
        local_copy.start()

        for off, slot in _OFF_TO_SLOT:
            dst = (my + off) % N_DEV
            rdma = pltpu.make_async_remote_copy(
                src_ref=x16_ref.at[pl.ds(dst * M_BLK, M_BLK), :],
                dst_ref=xg_ref.at[slot],
                send_sem=send_sems.at[off - 1],
                recv_sem=recv_sems.at[slot - 1],
                device_id=(dst,),
                device_id_type=pl.DeviceIdType.MESH,
            )
            rdma.start()
        local_copy.wait()

    for slot in (1, 2, 3):
        @pl.when((p == slot) & (j == 0))
        def _(slot=slot):
            recv = pltpu.make_async_remote_copy(
                src_ref=x16_ref.at[pl.ds(0, M_BLK), :],
                dst_ref=xg_ref.at[slot],
                send_sem=send_sems.at[0],
                recv_sem=recv_sems.at[slot - 1],
                device_id=(my,),
                device_id_type=pl.DeviceIdType.MESH,
            )
            recv.wait_recv()

    b = w_ref[...].astype(jnp.bfloat16)
    for s in range(N_DEV):
        @pl.when(p == s)
        def _(s=s):
            contrib = jnp.dot(
                xg_ref[s], b, preferred_element_type=jnp.float32
            )
            if s == 0:
                o_ref[...] = contrib
            elif s < N_DEV - 1:
                o_ref[...] = yin_ref[...] + contrib
            else:
                o_ref[...] = jnp.maximum(yin_ref[...] + contrib, 0.0)

    @pl.when((p == N_DEV - 1) & (j == NN2 - 1))
    def _():
        for off in (1, 2, 3):
            rdma = pltpu.make_async_remote_copy(
                src_ref=x16_ref.at[pl.ds(0, M_BLK), :],
                dst_ref=xg_ref.at[1],
                send_sem=send_sems.at[off - 1],
                recv_sem=recv_sems.at[0],
                device_id=(my,),
                device_id_type=pl.DeviceIdType.MESH,
            )
            rdma.wait_send()


def _fused(x16, w_mat, perm):
    grid_spec = pltpu.PrefetchScalarGridSpec(
        num_scalar_prefetch=1,
        grid=(N_DEV, NN2),
        in_specs=[
            pl.BlockSpec(memory_space=pl.ANY),
            pl.BlockSpec((K_PER, BN2), lambda p, j, perm_ref: (perm_ref[p], j)),
            pl.BlockSpec((M_BLK, BN2), lambda p, j, perm_ref: (0, j)),
        ],
        out_specs=pl.BlockSpec((M_BLK, BN2), lambda p, j, perm_ref: (0, j)),
        scratch_shapes=[
            pltpu.VMEM((N_DEV, M_BLK, K_PER), jnp.bfloat16),
            pltpu.SemaphoreType.DMA,
            pltpu.SemaphoreType.DMA((3,)),
            pltpu.SemaphoreType.DMA((3,)),
        ],
    )
    y0 = jnp.zeros((M_BLK, N_OUT), jnp.float32)
    return pl.pallas_call(
        _fused_body,
        grid_spec=grid_spec,
        out_shape=jax.ShapeDtypeStruct((M_BLK, N_OUT), jnp.float32),
        input_output_aliases={3: 0},
        compiler_params=pltpu.CompilerParams(
            dimension_semantics=("arbitrary", "arbitrary"),
            collective_id=0,
            vmem_limit_bytes=60 * 1024 * 1024,
        ),
    )(perm, x16, w_mat, y0)


def kernel(x, w_mat):
    my = lax.axis_index("i")
    x16 = x.astype(jnp.bfloat16)
    perm = (jnp.array([0, 3, 1, 2], jnp.int32) + my) % N_DEV
    return _fused(x16, w_mat, perm)
